# baseline (speedup 1.0000x reference)
import functools

import jax
import jax.numpy as jnp
from jax import lax
from jax.experimental import pallas as pl
from jax.experimental.pallas import tpu as pltpu
from jax.experimental.pallas import tpu_sc as plsc

N_USER = 5000
N_ITEM = 5000
N_NODES = N_USER + N_ITEM
D = 128
DP = 144
E = 100000
E_CORE = 150000
NC = 2
NS = 16
CHUNK = 128
SC_CHUNKS = 4
N_SUPER = 19
N_CHUNK = N_SUPER * SC_CHUNKS
E_TILE = N_CHUNK * CHUNK
E_PAD = NS * E_TILE
N_ACC = N_NODES
ROWS_TILE = N_ACC // NS


def _sc_segment_sums(table, src_idx, dst_idx, zrows):
  mesh = plsc.VectorSubcoreMesh(
      core_axis_name="c", subcore_axis_name="s", num_cores=NC, num_subcores=NS)

  @functools.partial(
      pl.kernel,
      out_type=jax.ShapeDtypeStruct((NC, N_ACC, DP), jnp.float32),
      mesh=mesh,
      scratch_types=[
          pltpu.VMEM_SHARED((N_ACC, DP), jnp.float32),
          pltpu.VMEM((3, SC_CHUNKS, CHUNK), jnp.int32),
          pltpu.VMEM((3, SC_CHUNKS, CHUNK), jnp.int32),
          pltpu.VMEM((CHUNK, DP), jnp.float32),
          pltpu.VMEM((CHUNK, DP), jnp.float32),
          pltpu.SemaphoreType.DMA,
          pltpu.SemaphoreType.DMA,
      ],
      compiler_params=pltpu.CompilerParams(use_tc_tiling_on_sc=False),
  )
  def kern(table_hbm, src_hbm, dst_hbm, zrows_hbm, out_hbm,
           acc, src_v, dst_v, rows_a, rows_b, sem_g, sem_i):
    c = lax.axis_index("c")
    s = lax.axis_index("s")

    pltpu.sync_copy(zrows_hbm, acc.at[pl.ds(s * ROWS_TILE, ROWS_TILE)])

    def issue_idx(k, b3):
      pltpu.async_copy(src_hbm.at[c, s, k], src_v.at[b3], sem_i)
      pltpu.async_copy(dst_hbm.at[c, s, k], dst_v.at[b3], sem_i)

    def wait_idx():
      pltpu.make_async_copy(src_hbm.at[0, 0, 0], src_v.at[0], sem_i).wait()
      pltpu.make_async_copy(dst_hbm.at[0, 0, 0], dst_v.at[0], sem_i).wait()

    def issue_gather(b3, m, buf):
      pltpu.async_copy(table_hbm.at[src_v.at[b3, m]], buf, sem_g)

    def wait_gather(buf):
      pltpu.make_async_copy(table_hbm.at[src_v.at[0, 0]], buf, sem_g).wait()

    def scatter(b3, m, buf):
      pltpu.sync_copy(buf, acc.at[dst_v.at[b3, m]], add=True)

    plsc.subcore_barrier()

    issue_idx(0, 0)
    wait_idx()
    issue_idx(1, 1)
    issue_gather(0, 0, rows_a)

    def body(k, carry):
      bk = lax.rem(k, 3)
      bn = lax.rem(k + 1, 3)
      rows = (rows_a, rows_b)
      for m in range(SC_CHUNKS - 1):
        issue_gather(bk, m + 1, rows[(m + 1) % 2])
        wait_gather(rows[m % 2])
        scatter(bk, m, rows[m % 2])

      @pl.when(k + 1 < N_SUPER)
      def _():
        wait_idx()

        @pl.when(k + 2 < N_SUPER)
        def _():
          issue_idx(k + 2, lax.rem(k + 2, 3))

        issue_gather(bn, 0, rows[SC_CHUNKS % 2])

      wait_gather(rows[(SC_CHUNKS - 1) % 2])
      scatter(bk, SC_CHUNKS - 1, rows[(SC_CHUNKS - 1) % 2])
      return carry

    lax.fori_loop(0, N_SUPER, body, 0, unroll=False)

    plsc.subcore_barrier()
    pltpu.sync_copy(acc.at[pl.ds(s * ROWS_TILE, ROWS_TILE)],
                    out_hbm.at[c, pl.ds(s * ROWS_TILE, ROWS_TILE)])

  return kern(table, src_idx, dst_idx, zrows)


def _tc_body(a_ref, v_ref, p0_ref, p1_ref, pb_ref, out_ref):
  i = pl.program_id(0)
  w = [a_ref[r, 0] * v_ref[0] + a_ref[r, 1] * v_ref[1] for r in range(3)]
  x0 = p0_ref[0]

  @pl.when(i < 5)
  def _user():
    s = x0[:, :D]
    d = jnp.maximum(x0[:, D:D + 1], 1.0)
    out_ref[...] = jnp.maximum(
        jnp.dot(s, w[0], preferred_element_type=jnp.float32) / d, 0.0)

  @pl.when(i >= 5)
  def _item():
    a_blk = x0 + p1_ref[0]
    b_blk = pb_ref[0]
    da = jnp.maximum(a_blk[:, D:D + 1], 1.0)
    db = jnp.maximum(b_blk[:, D:D + 1], 1.0)
    out_ref[...] = jnp.maximum(
        jnp.dot(a_blk[:, :D], w[1], preferred_element_type=jnp.float32) / da
        + jnp.dot(b_blk[:, :D], w[2], preferred_element_type=jnp.float32) / db,
        0.0)


def _tc_combine(partials, a, v):
  blk = 1000
  grid = (N_NODES // blk,)
  return pl.pallas_call(
      _tc_body,
      grid=grid,
      in_specs=[
          pl.BlockSpec((3, 2), lambda i: (0, 0), memory_space=pltpu.SMEM),
          pl.BlockSpec((2, D, D), lambda i: (0, 0, 0)),
          pl.BlockSpec((1, blk, DP), lambda i: (0, i, 0)),
          pl.BlockSpec((1, blk, DP), lambda i: (1, i, 0)),
          pl.BlockSpec((1, blk, DP), lambda i: (1, lax.rem(i, 5), 0)),
      ],
      out_specs=pl.BlockSpec((blk, D), lambda i: (i, 0)),
      out_shape=jax.ShapeDtypeStruct((N_NODES, D), jnp.float32),
  )(a, v, partials, partials, partials)


def kernel(embed_user, embed_item, V, a,
           edge_index_bought_by, edge_index_buys, edge_index_views):
  f32 = jnp.float32
  i32 = jnp.int32
  emb = jnp.concatenate([embed_user, embed_item], axis=0)
  table = jnp.concatenate(
      [emb, jnp.ones((N_NODES, 1), f32), jnp.zeros((N_NODES, DP - D - 1), f32)],
      axis=1)
  table = jnp.concatenate([table, jnp.zeros((1, DP), f32)], axis=0)

  half = E // 2
  bb_s, bb_d = edge_index_bought_by[0], edge_index_bought_by[1]
  by_s, by_d = edge_index_buys[0], edge_index_buys[1]
  vw_s, vw_d = edge_index_views[0], edge_index_views[1]
  src0 = jnp.concatenate([bb_s + N_USER, by_s[:half]])
  dst0 = jnp.concatenate([bb_d, by_d[:half] + N_USER])
  src1 = jnp.concatenate([vw_s, by_s[half:]])
  dst1 = jnp.concatenate([vw_d, by_d[half:] + N_USER])
  pad = E_PAD - E_CORE
  src_idx = jnp.concatenate(
      [jnp.stack([src0, src1]).astype(i32),
       jnp.full((NC, pad), N_NODES, i32)],
      axis=1).reshape(NC, NS, N_SUPER, SC_CHUNKS, CHUNK)
  dst_idx = jnp.concatenate(
      [jnp.stack([dst0, dst1]).astype(i32),
       jnp.zeros((NC, pad), i32)],
      axis=1).reshape(NC, NS, N_SUPER, SC_CHUNKS, CHUNK)

  zrows = jnp.zeros((ROWS_TILE, DP), f32)
  partials = _sc_segment_sums(table, src_idx, dst_idx, zrows)
  return _tc_combine(partials, a, V)


# speedup vs baseline: 1.9894x; 1.9894x over previous
import functools

import jax
import jax.numpy as jnp
from jax import lax
from jax.experimental import pallas as pl
from jax.experimental.pallas import tpu as pltpu
from jax.experimental.pallas import tpu_sc as plsc

N_USER = 5000
N_ITEM = 5000
N_NODES = N_USER + N_ITEM
D = 128
DW = 16
E = 100000
E_CORE = 150000
NC = 2
NS = 16
CHUNK = 64
N_CHUNK = 148
E_TILE = N_CHUNK * CHUNK
E_PAD = NS * E_TILE
N_ACC = 10016
ROWS_TILE = N_ACC // NS


def _sc_segment_sums(table, src_idx, dst_idx, zrows, zdeg, ones_rows):
  mesh = plsc.VectorSubcoreMesh(
      core_axis_name="c", subcore_axis_name="s", num_cores=NC, num_subcores=NS)

  @functools.partial(
      pl.kernel,
      out_type=(jax.ShapeDtypeStruct((NC, N_ACC, D), jnp.float32),
                jax.ShapeDtypeStruct((NC, N_ACC, DW), jnp.float32)),
      mesh=mesh,
      scratch_types=[
          pltpu.VMEM_SHARED((N_ACC, D), jnp.float32),
          pltpu.VMEM_SHARED((N_ACC, DW), jnp.float32),
          pltpu.VMEM((N_CHUNK, CHUNK), jnp.int32),
          pltpu.VMEM((N_CHUNK, CHUNK), jnp.int32),
          pltpu.VMEM((CHUNK, D), jnp.float32),
          pltpu.VMEM((CHUNK, D), jnp.float32),
          pltpu.VMEM((CHUNK, DW), jnp.float32),
          pltpu.SemaphoreType.DMA,
      ],
      compiler_params=pltpu.CompilerParams(use_tc_tiling_on_sc=False),
  )
  def kern(table_hbm, src_hbm, dst_hbm, zrows_hbm, zdeg_hbm, ones_hbm,
           out_hbm, deg_hbm, acc, deg, src_v, dst_v, rows_a, rows_b,
           ones_v, sem):
    c = lax.axis_index("c")
    s = lax.axis_index("s")

    pltpu.sync_copy(zrows_hbm, acc.at[pl.ds(s * ROWS_TILE, ROWS_TILE)])
    pltpu.sync_copy(zdeg_hbm, deg.at[pl.ds(s * ROWS_TILE, ROWS_TILE)])
    pltpu.sync_copy(ones_hbm, ones_v)
    pltpu.sync_copy(src_hbm.at[c, s], src_v)
    pltpu.sync_copy(dst_hbm.at[c, s], dst_v)
    plsc.subcore_barrier()

    def wait_chunk(j, buf):
      pltpu.make_async_copy(table_hbm.at[src_v.at[j]], buf, sem).wait()

    pltpu.async_copy(table_hbm.at[src_v.at[0]], rows_a, sem)

    def body(t, carry):
      j0 = 2 * t
      pltpu.async_copy(table_hbm.at[src_v.at[j0 + 1]], rows_b, sem)
      wait_chunk(j0, rows_a)
      pltpu.sync_copy(rows_a, acc.at[dst_v.at[j0]], add=True)
      pltpu.sync_copy(ones_v, deg.at[dst_v.at[j0]], add=True)

      @pl.when(t + 1 < N_CHUNK // 2)
      def _():
        pltpu.async_copy(table_hbm.at[src_v.at[j0 + 2]], rows_a, sem)

      wait_chunk(j0 + 1, rows_b)
      pltpu.sync_copy(rows_b, acc.at[dst_v.at[j0 + 1]], add=True)
      pltpu.sync_copy(ones_v, deg.at[dst_v.at[j0 + 1]], add=True)
      return carry

    lax.fori_loop(0, N_CHUNK // 2, body, 0, unroll=False)

    plsc.subcore_barrier()
    pltpu.sync_copy(acc.at[pl.ds(s * ROWS_TILE, ROWS_TILE)],
                    out_hbm.at[c, pl.ds(s * ROWS_TILE, ROWS_TILE)])
    pltpu.sync_copy(deg.at[pl.ds(s * ROWS_TILE, ROWS_TILE)],
                    deg_hbm.at[c, pl.ds(s * ROWS_TILE, ROWS_TILE)])

  return kern(table, src_idx, dst_idx, zrows, zdeg, ones_rows)


def _tc_body(a_ref, v_ref, p0_ref, p1_ref, pb_ref, d0_ref, d1_ref, db_ref,
             out_ref):
  i = pl.program_id(0)
  w = [a_ref[r, 0] * v_ref[0] + a_ref[r, 1] * v_ref[1] for r in range(3)]
  x0 = p0_ref[0]

  @pl.when(i < 5)
  def _user():
    d = jnp.maximum(d0_ref[0][:, :1], 1.0)
    out_ref[...] = jnp.maximum(
        jnp.dot(x0, w[0], preferred_element_type=jnp.float32) / d, 0.0)

  @pl.when(i >= 5)
  def _item():
    a_blk = x0 + p1_ref[0]
    da = jnp.maximum(d0_ref[0][:, :1] + d1_ref[0][:, :1], 1.0)
    db = jnp.maximum(db_ref[0][:, :1], 1.0)
    out_ref[...] = jnp.maximum(
        jnp.dot(a_blk, w[1], preferred_element_type=jnp.float32) / da
        + jnp.dot(pb_ref[0], w[2], preferred_element_type=jnp.float32) / db,
        0.0)


def _tc_combine(partials, degs, a, v):
  blk = 1000
  grid = (N_NODES // blk,)

  def pspec(fn):
    return pl.BlockSpec((1, blk, D), fn)

  def dspec(fn):
    return pl.BlockSpec((1, blk, DW), fn)

  return pl.pallas_call(
      _tc_body,
      grid=grid,
      in_specs=[
          pl.BlockSpec((3, 2), lambda i: (0, 0), memory_space=pltpu.SMEM),
          pl.BlockSpec((2, D, D), lambda i: (0, 0, 0)),
          pspec(lambda i: (0, i, 0)),
          pspec(lambda i: (1, i, 0)),
          pspec(lambda i: (1, lax.rem(i, 5), 0)),
          dspec(lambda i: (0, i, 0)),
          dspec(lambda i: (1, i, 0)),
          dspec(lambda i: (1, lax.rem(i, 5), 0)),
      ],
      out_specs=pl.BlockSpec((blk, D), lambda i: (i, 0)),
      out_shape=jax.ShapeDtypeStruct((N_NODES, D), jnp.float32),
  )(a, v, partials, partials, partials, degs, degs, degs)


def kernel(embed_user, embed_item, V, a,
           edge_index_bought_by, edge_index_buys, edge_index_views):
  f32 = jnp.float32
  i32 = jnp.int32
  table = jnp.concatenate(
      [embed_user, embed_item, jnp.zeros((1, D), f32)], axis=0)

  half = E // 2
  bb_s, bb_d = edge_index_bought_by[0], edge_index_bought_by[1]
  by_s, by_d = edge_index_buys[0], edge_index_buys[1]
  vw_s, vw_d = edge_index_views[0], edge_index_views[1]
  src0 = jnp.concatenate([bb_s + N_USER, by_s[:half]])
  dst0 = jnp.concatenate([bb_d, by_d[:half] + N_USER])
  src1 = jnp.concatenate([vw_s, by_s[half:]])
  dst1 = jnp.concatenate([vw_d, by_d[half:] + N_USER])
  pad = E_PAD - E_CORE
  src_idx = jnp.concatenate(
      [jnp.stack([src0, src1]).astype(i32),
       jnp.full((NC, pad), N_NODES, i32)], axis=1).reshape(NC, NS, N_CHUNK, CHUNK)
  dst_idx = jnp.concatenate(
      [jnp.stack([dst0, dst1]).astype(i32),
       jnp.full((NC, pad), N_NODES, i32)], axis=1).reshape(NC, NS, N_CHUNK, CHUNK)

  zrows = jnp.zeros((ROWS_TILE, D), f32)
  zdeg = jnp.zeros((ROWS_TILE, DW), f32)
  ones_rows = jnp.ones((CHUNK, DW), f32)
  partials, degs = _sc_segment_sums(table, src_idx, dst_idx, zrows, zdeg,
                                    ones_rows)
  return _tc_combine(partials, degs, a, V)


# speedup vs baseline: 1.9939x; 1.0023x over previous
import functools

import jax
import jax.numpy as jnp
from jax import lax
from jax.experimental import pallas as pl
from jax.experimental.pallas import tpu as pltpu
from jax.experimental.pallas import tpu_sc as plsc

N_USER = 5000
N_ITEM = 5000
N_NODES = N_USER + N_ITEM
D = 128
DW = 16
E = 100000
E_CORE = 150000
NC = 2
NS = 16
CHUNK = 64
N_CHUNK = 148
E_TILE = N_CHUNK * CHUNK
E_PAD = NS * E_TILE
N_ACC = 10016
ROWS_TILE = N_ACC // NS


def _sc_segment_sums(table, src_idx, dst_idx, zrows, zdeg, ones_rows):
  mesh = plsc.VectorSubcoreMesh(
      core_axis_name="c", subcore_axis_name="s", num_cores=NC, num_subcores=NS)

  @functools.partial(
      pl.kernel,
      out_type=(jax.ShapeDtypeStruct((NC, N_ACC, D), jnp.float32),
                jax.ShapeDtypeStruct((NC, N_ACC, DW), jnp.float32)),
      mesh=mesh,
      scratch_types=[
          pltpu.VMEM_SHARED((N_ACC, D), jnp.float32),
          pltpu.VMEM_SHARED((N_ACC, DW), jnp.float32),
          pltpu.VMEM((N_CHUNK, CHUNK), jnp.int32),
          pltpu.VMEM((N_CHUNK, CHUNK), jnp.int32),
          pltpu.VMEM((CHUNK, D), jnp.float32),
          pltpu.VMEM((CHUNK, D), jnp.float32),
          pltpu.VMEM((CHUNK, DW), jnp.float32),
          pltpu.SemaphoreType.DMA,
          pltpu.SemaphoreType.DMA,
      ],
      compiler_params=pltpu.CompilerParams(use_tc_tiling_on_sc=False),
  )
  def kern(table_hbm, src_hbm, dst_hbm, zrows_hbm, zdeg_hbm, ones_hbm,
           out_hbm, deg_hbm, acc, deg, src_v, dst_v, rows_a, rows_b,
           ones_v, sem, sem_d):
    c = lax.axis_index("c")
    s = lax.axis_index("s")

    pltpu.sync_copy(zrows_hbm, acc.at[pl.ds(s * ROWS_TILE, ROWS_TILE)])
    pltpu.sync_copy(zdeg_hbm, deg.at[pl.ds(s * ROWS_TILE, ROWS_TILE)])
    pltpu.sync_copy(ones_hbm, ones_v)
    pltpu.sync_copy(src_hbm.at[c, s], src_v)
    pltpu.sync_copy(dst_hbm.at[c, s], dst_v)
    plsc.subcore_barrier()

    def wait_chunk(j, buf):
      pltpu.make_async_copy(table_hbm.at[src_v.at[j]], buf, sem).wait()

    pltpu.async_copy(table_hbm.at[src_v.at[0]], rows_a, sem)

    def body(t, carry):
      j0 = 2 * t
      pltpu.async_copy(table_hbm.at[src_v.at[j0 + 1]], rows_b, sem)
      wait_chunk(j0, rows_a)
      pltpu.sync_copy(rows_a, acc.at[dst_v.at[j0]], add=True)

      @pl.when(t + 1 < N_CHUNK // 2)
      def _():
        pltpu.async_copy(table_hbm.at[src_v.at[j0 + 2]], rows_a, sem)

      wait_chunk(j0 + 1, rows_b)
      pltpu.sync_copy(rows_b, acc.at[dst_v.at[j0 + 1]], add=True)
      return carry

    lax.fori_loop(0, N_CHUNK // 2, body, 0, unroll=False)

    DEPTH = 8

    def wait_deg():
      pltpu.make_async_copy(zdeg_hbm.at[pl.ds(0, CHUNK)], ones_v, sem_d).wait()

    def deg_body(j, carry):
      pltpu.async_copy(ones_v, deg.at[dst_v.at[j]], sem_d, add=True)

      @pl.when(j >= DEPTH)
      def _():
        wait_deg()

      return carry

    lax.fori_loop(0, N_CHUNK, deg_body, 0, unroll=False)
    for _ in range(DEPTH):
      wait_deg()

    plsc.subcore_barrier()
    pltpu.sync_copy(acc.at[pl.ds(s * ROWS_TILE, ROWS_TILE)],
                    out_hbm.at[c, pl.ds(s * ROWS_TILE, ROWS_TILE)])
    pltpu.sync_copy(deg.at[pl.ds(s * ROWS_TILE, ROWS_TILE)],
                    deg_hbm.at[c, pl.ds(s * ROWS_TILE, ROWS_TILE)])

  return kern(table, src_idx, dst_idx, zrows, zdeg, ones_rows)


def _tc_body(a_ref, v_ref, p0_ref, p1_ref, pb_ref, d0_ref, d1_ref, db_ref,
             out_ref):
  i = pl.program_id(0)
  w = [a_ref[r, 0] * v_ref[0] + a_ref[r, 1] * v_ref[1] for r in range(3)]
  x0 = p0_ref[0]

  @pl.when(i < 5)
  def _user():
    d = jnp.maximum(d0_ref[0][:, :1], 1.0)
    out_ref[...] = jnp.maximum(
        jnp.dot(x0, w[0], preferred_element_type=jnp.float32) / d, 0.0)

  @pl.when(i >= 5)
  def _item():
    a_blk = x0 + p1_ref[0]
    da = jnp.maximum(d0_ref[0][:, :1] + d1_ref[0][:, :1], 1.0)
    db = jnp.maximum(db_ref[0][:, :1], 1.0)
    out_ref[...] = jnp.maximum(
        jnp.dot(a_blk, w[1], preferred_element_type=jnp.float32) / da
        + jnp.dot(pb_ref[0], w[2], preferred_element_type=jnp.float32) / db,
        0.0)


def _tc_combine(partials, degs, a, v):
  blk = 1000
  grid = (N_NODES // blk,)

  def pspec(fn):
    return pl.BlockSpec((1, blk, D), fn)

  def dspec(fn):
    return pl.BlockSpec((1, blk, DW), fn)

  return pl.pallas_call(
      _tc_body,
      grid=grid,
      in_specs=[
          pl.BlockSpec((3, 2), lambda i: (0, 0), memory_space=pltpu.SMEM),
          pl.BlockSpec((2, D, D), lambda i: (0, 0, 0)),
          pspec(lambda i: (0, i, 0)),
          pspec(lambda i: (1, i, 0)),
          pspec(lambda i: (1, lax.rem(i, 5), 0)),
          dspec(lambda i: (0, i, 0)),
          dspec(lambda i: (1, i, 0)),
          dspec(lambda i: (1, lax.rem(i, 5), 0)),
      ],
      out_specs=pl.BlockSpec((blk, D), lambda i: (i, 0)),
      out_shape=jax.ShapeDtypeStruct((N_NODES, D), jnp.float32),
  )(a, v, partials, partials, partials, degs, degs, degs)


def kernel(embed_user, embed_item, V, a,
           edge_index_bought_by, edge_index_buys, edge_index_views):
  f32 = jnp.float32
  i32 = jnp.int32
  table = jnp.concatenate(
      [embed_user, embed_item, jnp.zeros((1, D), f32)], axis=0)

  half = E // 2
  bb_s, bb_d = edge_index_bought_by[0], edge_index_bought_by[1]
  by_s, by_d = edge_index_buys[0], edge_index_buys[1]
  vw_s, vw_d = edge_index_views[0], edge_index_views[1]
  src0 = jnp.concatenate([bb_s + N_USER, by_s[:half]])
  dst0 = jnp.concatenate([bb_d, by_d[:half] + N_USER])
  src1 = jnp.concatenate([vw_s, by_s[half:]])
  dst1 = jnp.concatenate([vw_d, by_d[half:] + N_USER])
  pad = E_PAD - E_CORE
  src_idx = jnp.concatenate(
      [jnp.stack([src0, src1]).astype(i32),
       jnp.full((NC, pad), N_NODES, i32)], axis=1).reshape(NC, NS, N_CHUNK, CHUNK)
  dst_idx = jnp.concatenate(
      [jnp.stack([dst0, dst1]).astype(i32),
       jnp.full((NC, pad), N_NODES, i32)], axis=1).reshape(NC, NS, N_CHUNK, CHUNK)

  zrows = jnp.zeros((ROWS_TILE, D), f32)
  zdeg = jnp.zeros((ROWS_TILE, DW), f32)
  ones_rows = jnp.ones((CHUNK, DW), f32)
  partials, degs = _sc_segment_sums(table, src_idx, dst_idx, zrows, zdeg,
                                    ones_rows)
  return _tc_combine(partials, degs, a, V)
